# Initial kernel scaffold; baseline (speedup 1.0000x reference)
import jax
import jax.numpy as jnp
from jax.experimental import pallas as pl


def kernel(x, cc_mask, edge_index, W1L, W1H, W2L, W2H, WX, lam1, lam2, lin_w, lin_b):
    raise NotImplementedError("write your pallas kernel here")


# speedup vs baseline: 17.8281x; 17.8281x over previous
import functools
import jax
import jax.numpy as jnp
from jax import lax
from jax.experimental import pallas as pl
from jax.experimental.pallas import tpu as pltpu
from jax.experimental.pallas import tpu_sc as plsc

N = 10000
F = 128
H = 256
C = 40
NP = 10240
E = 320000
CK = 128
EP = 327680
EG = EP // CK
ZR = 2 * NP
TH = 2 * NP + 16
NTILES = 32
CH1 = EG // NTILES
CH2 = EG // 16
RPT = NP // 16

_mesh = plsc.VectorSubcoreMesh(core_axis_name="c", subcore_axis_name="s")


def _zero_rows(buf, nrows):
    z = jnp.zeros((16,), jnp.float32)
    for i in range(nrows):
        for k in range(8):
            buf[i, pl.ds(k * 16, 16)] = z


@functools.partial(
    pl.kernel,
    out_type=(
        jax.ShapeDtypeStruct((EG, CK), jnp.int32),
        jax.ShapeDtypeStruct((EG, CK), jnp.int32),
        jax.ShapeDtypeStruct((2, NP), jnp.float32),
        jax.ShapeDtypeStruct((2, NP), jnp.float32),
    ),
    mesh=_mesh,
    scratch_types=[
        pltpu.VMEM((CH1, CK), jnp.int32),
        pltpu.VMEM((CH1, CK), jnp.int32),
        pltpu.VMEM((CK,), jnp.float32),
        pltpu.VMEM((CK,), jnp.float32),
        pltpu.VMEM((CH1, CK), jnp.int32),
        pltpu.VMEM((CH1, CK), jnp.int32),
        pltpu.VMEM((CH1, CK), jnp.float32),
        pltpu.VMEM((CH1, CK), jnp.float32),
        pltpu.VMEM((RPT,), jnp.float32),
        pltpu.VMEM_SHARED((NP,), jnp.float32),
        pltpu.VMEM_SHARED((NP,), jnp.float32),
    ],
)
def _prep(rows_h, cols_h, cc_h, g1_h, g2_h, degb_h, degc_h,
          rows_v, cols_v, ccr_v, ccc_v, g1_v, g2_v, v1_v, v2_v, zb,
          db_sh, dc_sh):
    c = lax.axis_index("c")
    s = lax.axis_index("s")
    w = c * 16 + s
    pltpu.sync_copy(rows_h.at[pl.ds(w * CH1, CH1)], rows_v)
    pltpu.sync_copy(cols_h.at[pl.ds(w * CH1, CH1)], cols_v)
    z = jnp.zeros((16,), jnp.float32)
    for i in range(RPT // 16):
        zb[pl.ds(i * 16, 16)] = z
    pltpu.sync_copy(zb, db_sh.at[pl.ds(s * RPT, RPT)])
    pltpu.sync_copy(zb, dc_sh.at[pl.ds(s * RPT, RPT)])
    plsc.subcore_barrier()

    def body(j, _):
        pltpu.sync_copy(cc_h.at[rows_v.at[j]], ccr_v)
        pltpu.sync_copy(cc_h.at[cols_v.at[j]], ccc_v)
        for k in range(8):
            sl = pl.ds(k * 16, 16)
            r = rows_v[j, sl]
            cl = cols_v[j, sl]
            nonself = r != cl
            ccr = ccr_v[sl]
            ccc = ccc_v[sl]
            v1 = jnp.where(nonself, 1.0, 0.0).astype(jnp.float32)
            v1_v[j, sl] = v1
            v2_v[j, sl] = v1 * ccc
            sel1 = ccr.astype(jnp.int32)
            sel2 = ccc.astype(jnp.int32)
            g1_v[j, sl] = jnp.where(nonself, sel1 * NP + cl, ZR)
            g2_v[j, sl] = jnp.where(nonself, sel2 * NP + cl, ZR)
        pltpu.sync_copy(v1_v.at[j], db_sh.at[rows_v.at[j]], add=True)
        pltpu.sync_copy(v2_v.at[j], dc_sh.at[rows_v.at[j]], add=True)
        return _

    lax.fori_loop(0, CH1, body, None)
    pltpu.sync_copy(g1_v, g1_h.at[pl.ds(w * CH1, CH1)])
    pltpu.sync_copy(g2_v, g2_h.at[pl.ds(w * CH1, CH1)])
    plsc.subcore_barrier()
    pltpu.sync_copy(db_sh.at[pl.ds(s * RPT, RPT)], zb)
    pltpu.sync_copy(zb, degb_h.at[c, pl.ds(s * RPT, RPT)])
    pltpu.sync_copy(dc_sh.at[pl.ds(s * RPT, RPT)], zb)
    pltpu.sync_copy(zb, degc_h.at[c, pl.ds(s * RPT, RPT)])


SLAB1 = CH1 // 2


@functools.partial(
    pl.kernel,
    out_type=jax.ShapeDtypeStruct((2, NP, F), jnp.float32),
    mesh=_mesh,
    scratch_types=[
        pltpu.VMEM((SLAB1, CK), jnp.int32),
        pltpu.VMEM((SLAB1, CK), jnp.int32),
        pltpu.VMEM((CK, F), jnp.float32),
        pltpu.VMEM((CK, F), jnp.float32),
        pltpu.VMEM_SHARED((NP, F), jnp.float32),
        pltpu.SemaphoreType.DMA,
        pltpu.SemaphoreType.DMA,
    ],
)
def _spmm1(g1_h, rows_h, t1_h, out_h, g1_v, rows_v, buf0, buf1, acc_sh,
           sem0, sem1):
    c = lax.axis_index("c")
    s = lax.axis_index("s")
    w = c * 16 + s
    _zero_rows(buf0, CK)
    for t in range(RPT // CK):
        pltpu.sync_copy(buf0, acc_sh.at[pl.ds(s * RPT + t * CK, CK)])
    plsc.subcore_barrier()

    def body(i, _):
        j = i * 2
        h0 = pltpu.async_copy(t1_h.at[g1_v.at[j]], buf0, sem0)
        h1 = pltpu.async_copy(t1_h.at[g1_v.at[j + 1]], buf1, sem1)
        h0.wait()
        pltpu.sync_copy(buf0, acc_sh.at[rows_v.at[j]], add=True)
        h1.wait()
        pltpu.sync_copy(buf1, acc_sh.at[rows_v.at[j + 1]], add=True)
        return _

    for p in range(CH1 // SLAB1):
        base = w * CH1 + p * SLAB1
        pltpu.sync_copy(g1_h.at[pl.ds(base, SLAB1)], g1_v)
        pltpu.sync_copy(rows_h.at[pl.ds(base, SLAB1)], rows_v)
        lax.fori_loop(0, SLAB1 // 2, body, None)
    plsc.subcore_barrier()
    for t in range(RPT // CK):
        sl = pl.ds(s * RPT + t * CK, CK)
        pltpu.sync_copy(acc_sh.at[sl], buf0)
        pltpu.sync_copy(buf0, out_h.at[c, sl])


SLAB2 = CH2 // 4


@functools.partial(
    pl.kernel,
    out_type=jax.ShapeDtypeStruct((2, NP, F), jnp.float32),
    mesh=_mesh,
    scratch_types=[
        pltpu.VMEM((SLAB2, CK), jnp.int32),
        pltpu.VMEM((SLAB2, CK), jnp.int32),
        pltpu.VMEM((SLAB2, CK), jnp.int32),
        pltpu.VMEM((CK,), jnp.float32),
        pltpu.VMEM((CK,), jnp.float32),
        pltpu.VMEM((CK, F), jnp.float32),
        pltpu.VMEM((CK, F), jnp.float32),
        pltpu.VMEM_SHARED((NP, F), jnp.float32),
        pltpu.SemaphoreType.DMA,
        pltpu.SemaphoreType.DMA,
    ],
)
def _spmm2(g2_h, rows_h, a2_h, t2_h, out_h, g2_v, rows_v, r2i_v, r2b0, r2b1,
           buf0, buf1, acc_sh, sem0, sem1):
    c = lax.axis_index("c")
    s = lax.axis_index("s")
    off = c * TH

    def initbody(j, _):
        for k in range(8):
            sl = pl.ds(k * 16, 16)
            g2 = g2_v[j, sl]
            r2i_v[j, sl] = jnp.where(g2 >= NP, NP, 0) + rows_v[j, sl]
            g2_v[j, sl] = g2 + off
        return _

    _zero_rows(buf0, CK)
    for t in range(RPT // CK):
        pltpu.sync_copy(buf0, acc_sh.at[pl.ds(s * RPT + t * CK, CK)])
    plsc.subcore_barrier()

    def scale(buf, r2b):
        def rb(g, _):
            w16 = r2b[pl.ds(g * 16, 16)]
            for b in range(16):
                wv = w16[b]
                i = g * 16 + b
                for k in range(8):
                    sl = pl.ds(k * 16, 16)
                    buf[i, sl] = buf[i, sl] * wv
            return _
        lax.fori_loop(0, CK // 16, rb, None)

    def body(i, _):
        j = i * 2
        h0 = pltpu.async_copy(t2_h.at[g2_v.at[j]], buf0, sem0)
        h1 = pltpu.async_copy(t2_h.at[g2_v.at[j + 1]], buf1, sem1)
        pltpu.sync_copy(a2_h.at[r2i_v.at[j]], r2b0)
        pltpu.sync_copy(a2_h.at[r2i_v.at[j + 1]], r2b1)
        h0.wait()
        scale(buf0, r2b0)
        pltpu.sync_copy(buf0, acc_sh.at[rows_v.at[j]], add=True)
        h1.wait()
        scale(buf1, r2b1)
        pltpu.sync_copy(buf1, acc_sh.at[rows_v.at[j + 1]], add=True)
        return _

    for p in range(CH2 // SLAB2):
        base = s * CH2 + p * SLAB2
        pltpu.sync_copy(g2_h.at[pl.ds(base, SLAB2)], g2_v)
        pltpu.sync_copy(rows_h.at[pl.ds(base, SLAB2)], rows_v)
        lax.fori_loop(0, SLAB2, initbody, None)
        lax.fori_loop(0, SLAB2 // 2, body, None)
    plsc.subcore_barrier()
    for t in range(RPT // CK):
        sl = pl.ds(s * RPT + t * CK, CK)
        pltpu.sync_copy(acc_sh.at[sl], buf0)
        pltpu.sync_copy(buf0, out_h.at[c, sl])


BM = 256


def _tca_body(x_b, cc_b, db_b, dc_b, orev, occ, a1_o, a2cc_o, a2rev_o):
    db = db_b[0] + db_b[1]
    dc = dc_b[0] + dc_b[1]
    cc = cc_b[...]
    x = x_b[...]
    a1_o[...] = lax.rsqrt(db + 1.0)
    occ[...] = x * lax.rsqrt(cc * db + 1.0)
    orev[...] = x * lax.rsqrt((1.0 - cc) * db + 1.0)
    a2cc_o[...] = lax.rsqrt(dc + 1.0)
    a2rev_o[...] = lax.rsqrt(db - dc + 1.0)


def _tcb_body(x_b, cc_b, a1_b, a2cc_b, a2rev_b, lamx_b, acc_b,
              w1l, w1h, w2l, w2h, wx, t2rev_o, t2cc_o, rest_o):
    x = x_b[...]
    cc = cc_b[...]
    a1 = a1_b[...]
    agg1 = acc_b[0] + acc_b[1]
    selden = a1 * agg1 + (a1 * a1) * x
    xlin = jnp.where(cc > 0.0, selden, x)
    xhin = jnp.where(cc > 0.0, x, selden)
    xl = jnp.maximum(jnp.dot(xlin, w1l[...], preferred_element_type=jnp.float32), 0.0)
    xh = jnp.maximum(jnp.dot(xhin, w1h[...], preferred_element_type=jnp.float32), 0.0)
    yl = jnp.dot(xl, w2l[...], preferred_element_type=jnp.float32)
    yh = jnp.dot(xh, w2h[...], preferred_element_type=jnp.float32)
    a2cc = a2cc_b[...]
    a2rev = a2rev_b[...]
    t2cc_o[...] = a2cc * yl
    t2rev_o[...] = a2rev * yh
    rest_o[...] = (lamx_b[...] * jnp.dot(x, wx[...], preferred_element_type=jnp.float32)
                   + (a2cc * a2cc) * yl + (a2rev * a2rev) * yh)


def _tcc_body(a2a_b, a2b_b, rest_b, lw_b, lb_b, out_o):
    rest = rest_b[...]
    xf_lo = jnp.maximum(a2a_b[...] + rest[:, :F], 0.0)
    xf_hi = jnp.maximum(a2b_b[...] + rest[:, F:], 0.0)
    lw = lw_b[...]
    out_o[...] = (jnp.dot(xf_lo, lw[:F, :], preferred_element_type=jnp.float32)
                  + jnp.dot(xf_hi, lw[F:, :], preferred_element_type=jnp.float32)
                  + lb_b[...])


def _col_spec(bm, width):
    return pl.BlockSpec((bm, width), lambda i: (i, 0))


def kernel(x, cc_mask, edge_index, W1L, W1H, W2L, W2H, WX, lam1, lam2, lin_w, lin_b):
    f32 = jnp.float32
    ei = edge_index.astype(jnp.int32)
    rows = jnp.pad(ei[1], (0, EP - E), constant_values=N).reshape(EG, CK)
    cols = jnp.pad(ei[0], (0, EP - E), constant_values=N).reshape(EG, CK)
    cc_p = jnp.pad(cc_mask.astype(f32), (0, NP - N))
    x_p = jnp.pad(x.astype(f32), ((0, NP - N), (0, 0)))

    g1, g2, degb_p, degc_p = _prep(rows, cols, cc_p)

    nblocks = NP // BM
    vec_spec = _col_spec(BM, 1)
    deg_spec = pl.BlockSpec((2, BM, 1), lambda i: (0, i, 0))
    orev, occ, a1, a2cc, a2rev = pl.pallas_call(
        _tca_body,
        grid=(nblocks,),
        in_specs=[_col_spec(BM, F), vec_spec, deg_spec, deg_spec],
        out_specs=[_col_spec(BM, F), _col_spec(BM, F), vec_spec, vec_spec, vec_spec],
        out_shape=[
            jax.ShapeDtypeStruct((NP, F), f32),
            jax.ShapeDtypeStruct((NP, F), f32),
            jax.ShapeDtypeStruct((NP, 1), f32),
            jax.ShapeDtypeStruct((NP, 1), f32),
            jax.ShapeDtypeStruct((NP, 1), f32),
        ],
    )(x_p, cc_p.reshape(NP, 1), degb_p.reshape(2, NP, 1), degc_p.reshape(2, NP, 1))

    t1 = jnp.concatenate([orev, occ, jnp.zeros((16, F), f32)], axis=0)
    acc1 = _spmm1(g1, rows, t1)

    l1 = jax.nn.softmax(lam1)
    l2 = jax.nn.softmax(lam2)
    lamx = (l1[0] * cc_p + l2[0] * (1.0 - cc_p)).reshape(NP, 1)
    w2l_s = W2L.astype(f32) * l1[1]
    w2h_s = W2H.astype(f32) * l2[1]

    full = lambda shape: pl.BlockSpec(shape, lambda i: tuple(0 for _ in shape))
    t2rev, t2cc, rest = pl.pallas_call(
        _tcb_body,
        grid=(nblocks,),
        in_specs=[
            _col_spec(BM, F), vec_spec, vec_spec, vec_spec, vec_spec, vec_spec,
            pl.BlockSpec((2, BM, F), lambda i: (0, i, 0)),
            full((F, H)), full((F, H)), full((H, H)), full((H, H)), full((F, H)),
        ],
        out_specs=[_col_spec(BM, H), _col_spec(BM, H), _col_spec(BM, H)],
        out_shape=[
            jax.ShapeDtypeStruct((NP, H), f32),
            jax.ShapeDtypeStruct((NP, H), f32),
            jax.ShapeDtypeStruct((NP, H), f32),
        ],
    )(x_p, cc_p.reshape(NP, 1), a1, a2cc, a2rev, lamx, acc1,
      W1L.astype(f32), W1H.astype(f32), w2l_s, w2h_s, WX.astype(f32))

    z16 = jnp.zeros((16, F), f32)
    t2s = jnp.concatenate(
        [t2rev[:, :F], t2cc[:, :F], z16, t2rev[:, F:], t2cc[:, F:], z16], axis=0)
    a2vec = jnp.concatenate([a2rev[:, 0], a2cc[:, 0], jnp.zeros((16,), f32)])

    acc2 = _spmm2(g2, rows, a2vec, t2s)

    lw_pad = jnp.pad(lin_w.astype(f32), ((0, 0), (0, 128 - C)))
    lb_pad = jnp.pad(lin_b.astype(f32), (0, 128 - C)).reshape(1, 128)
    outp = pl.pallas_call(
        _tcc_body,
        grid=(nblocks,),
        in_specs=[_col_spec(BM, F), _col_spec(BM, F), _col_spec(BM, H),
                  full((H, 128)), full((1, 128))],
        out_specs=_col_spec(BM, 128),
        out_shape=jax.ShapeDtypeStruct((NP, 128), f32),
    )(acc2[0], acc2[1], rest, lw_pad, lb_pad)

    return outp[:N, :C]


# speedup vs baseline: 18.3149x; 1.0273x over previous
import functools
import jax
import jax.numpy as jnp
from jax import lax
from jax.experimental import pallas as pl
from jax.experimental.pallas import tpu as pltpu
from jax.experimental.pallas import tpu_sc as plsc

N = 10000
F = 128
H = 256
C = 40
NP = 10240
E = 320000
CK = 128
EP = 327680
EG = EP // CK
ZR = 2 * NP
TH = 2 * NP + 16
NTILES = 32
CH1 = EG // NTILES
CH2 = EG // 16
RPT = NP // 16

_mesh = plsc.VectorSubcoreMesh(core_axis_name="c", subcore_axis_name="s")


def _zero_rows(buf, nrows, ncols=128):
    z = jnp.zeros((16,), jnp.float32)
    for i in range(nrows):
        for k in range(ncols // 16):
            buf[i, pl.ds(k * 16, 16)] = z


@functools.partial(
    pl.kernel,
    out_type=(
        jax.ShapeDtypeStruct((EG, CK), jnp.int32),
        jax.ShapeDtypeStruct((EG, CK), jnp.int32),
        jax.ShapeDtypeStruct((2, NP), jnp.float32),
        jax.ShapeDtypeStruct((2, NP), jnp.float32),
    ),
    mesh=_mesh,
    scratch_types=[
        pltpu.VMEM((CH1, CK), jnp.int32),
        pltpu.VMEM((CH1, CK), jnp.int32),
        pltpu.VMEM((CK,), jnp.float32),
        pltpu.VMEM((CK,), jnp.float32),
        pltpu.VMEM((CH1, CK), jnp.int32),
        pltpu.VMEM((CH1, CK), jnp.int32),
        pltpu.VMEM((CH1, CK), jnp.float32),
        pltpu.VMEM((CH1, CK), jnp.float32),
        pltpu.VMEM((RPT,), jnp.float32),
        pltpu.VMEM_SHARED((NP,), jnp.float32),
        pltpu.VMEM_SHARED((NP,), jnp.float32),
    ],
)
def _prep(rows_h, cols_h, cc_h, g1_h, g2_h, degb_h, degc_h,
          rows_v, cols_v, ccr_v, ccc_v, g1_v, g2_v, v1_v, v2_v, zb,
          db_sh, dc_sh):
    c = lax.axis_index("c")
    s = lax.axis_index("s")
    w = c * 16 + s
    pltpu.sync_copy(rows_h.at[pl.ds(w * CH1, CH1)], rows_v)
    pltpu.sync_copy(cols_h.at[pl.ds(w * CH1, CH1)], cols_v)
    z = jnp.zeros((16,), jnp.float32)
    for i in range(RPT // 16):
        zb[pl.ds(i * 16, 16)] = z
    pltpu.sync_copy(zb, db_sh.at[pl.ds(s * RPT, RPT)])
    pltpu.sync_copy(zb, dc_sh.at[pl.ds(s * RPT, RPT)])
    plsc.subcore_barrier()

    def body(j, _):
        pltpu.sync_copy(cc_h.at[rows_v.at[j]], ccr_v)
        pltpu.sync_copy(cc_h.at[cols_v.at[j]], ccc_v)
        for k in range(8):
            sl = pl.ds(k * 16, 16)
            r = rows_v[j, sl]
            cl = cols_v[j, sl]
            nonself = r != cl
            ccr = ccr_v[sl]
            ccc = ccc_v[sl]
            v1 = jnp.where(nonself, 1.0, 0.0).astype(jnp.float32)
            v1_v[j, sl] = v1
            v2_v[j, sl] = v1 * ccc
            sel1 = ccr.astype(jnp.int32)
            sel2 = ccc.astype(jnp.int32)
            g1_v[j, sl] = jnp.where(nonself, sel1 * NP + cl, ZR)
            g2_v[j, sl] = jnp.where(nonself, sel2 * NP + cl, ZR)
        pltpu.sync_copy(v1_v.at[j], db_sh.at[rows_v.at[j]], add=True)
        pltpu.sync_copy(v2_v.at[j], dc_sh.at[rows_v.at[j]], add=True)
        return _

    lax.fori_loop(0, CH1, body, None)
    pltpu.sync_copy(g1_v, g1_h.at[pl.ds(w * CH1, CH1)])
    pltpu.sync_copy(g2_v, g2_h.at[pl.ds(w * CH1, CH1)])
    plsc.subcore_barrier()
    pltpu.sync_copy(db_sh.at[pl.ds(s * RPT, RPT)], zb)
    pltpu.sync_copy(zb, degb_h.at[c, pl.ds(s * RPT, RPT)])
    pltpu.sync_copy(dc_sh.at[pl.ds(s * RPT, RPT)], zb)
    pltpu.sync_copy(zb, degc_h.at[c, pl.ds(s * RPT, RPT)])


SLAB1 = CH1 // 2


@functools.partial(
    pl.kernel,
    out_type=jax.ShapeDtypeStruct((2, NP, F), jnp.float32),
    mesh=_mesh,
    scratch_types=[
        pltpu.VMEM((SLAB1, CK), jnp.int32),
        pltpu.VMEM((SLAB1, CK), jnp.int32),
        pltpu.VMEM((CK, F), jnp.float32),
        pltpu.VMEM((CK, F), jnp.float32),
        pltpu.VMEM_SHARED((NP, F), jnp.float32),
        pltpu.SemaphoreType.DMA,
        pltpu.SemaphoreType.DMA,
    ],
)
def _spmm1(g1_h, rows_h, t1_h, out_h, g1_v, rows_v, buf0, buf1, acc_sh,
           sem0, sem1):
    c = lax.axis_index("c")
    s = lax.axis_index("s")
    w = c * 16 + s
    _zero_rows(buf0, CK)
    for t in range(RPT // CK):
        pltpu.sync_copy(buf0, acc_sh.at[pl.ds(s * RPT + t * CK, CK)])
    plsc.subcore_barrier()

    def body(i, _):
        j = i * 2
        h0 = pltpu.async_copy(t1_h.at[g1_v.at[j]], buf0, sem0)
        h1 = pltpu.async_copy(t1_h.at[g1_v.at[j + 1]], buf1, sem1)
        h0.wait()
        pltpu.sync_copy(buf0, acc_sh.at[rows_v.at[j]], add=True)
        h1.wait()
        pltpu.sync_copy(buf1, acc_sh.at[rows_v.at[j + 1]], add=True)
        return _

    for p in range(CH1 // SLAB1):
        base = w * CH1 + p * SLAB1
        pltpu.sync_copy(g1_h.at[pl.ds(base, SLAB1)], g1_v)
        pltpu.sync_copy(rows_h.at[pl.ds(base, SLAB1)], rows_v)
        lax.fori_loop(0, SLAB1 // 2, body, None)
    plsc.subcore_barrier()
    for t in range(RPT // CK):
        sl = pl.ds(s * RPT + t * CK, CK)
        pltpu.sync_copy(acc_sh.at[sl], buf0)
        pltpu.sync_copy(buf0, out_h.at[c, sl])


SLAB2 = CH2 // 4
FS = F // 2
SH = 4 * TH
RPT2 = 2 * NP // 16


@functools.partial(
    pl.kernel,
    out_type=jax.ShapeDtypeStruct((4, 2 * NP, FS), jnp.float32),
    mesh=_mesh,
    compiler_params=pltpu.CompilerParams(use_tc_tiling_on_sc=False),
    scratch_types=[
        pltpu.VMEM((SLAB2, CK), jnp.int32),
        pltpu.VMEM((SLAB2, CK), jnp.int32),
        pltpu.VMEM((CK, FS), jnp.float32),
        pltpu.VMEM((CK, FS), jnp.float32),
        pltpu.VMEM_SHARED((2 * NP, FS), jnp.float32),
        pltpu.SemaphoreType.DMA,
        pltpu.SemaphoreType.DMA,
    ],
)
def _spmm2(g2_h, rows_h, t2_h, out_h, g2_v, sidx_v, buf0, buf1, acc_sh,
           sem0, sem1):
    c = lax.axis_index("c")
    s = lax.axis_index("s")

    def mainbody(i, _):
        j = i * 2
        h0 = pltpu.async_copy(t2_h.at[g2_v.at[j]], buf0, sem0)
        h1 = pltpu.async_copy(t2_h.at[g2_v.at[j + 1]], buf1, sem1)
        h0.wait()
        pltpu.sync_copy(buf0, acc_sh.at[sidx_v.at[j]], add=True)
        h1.wait()
        pltpu.sync_copy(buf1, acc_sh.at[sidx_v.at[j + 1]], add=True)
        return _

    for p in range(2):
        strip = 2 * c + p
        off = strip * TH

        def initbody(j, _, off=off):
            for k in range(8):
                sl = pl.ds(k * 16, 16)
                g2 = g2_v[j, sl]
                sidx_v[j, sl] = jnp.where(g2 >= NP, NP, 0) + sidx_v[j, sl]
                g2_v[j, sl] = g2 + off
            return _

        _zero_rows(buf0, CK, FS)
        for t in range(RPT2 // CK):
            pltpu.sync_copy(buf0, acc_sh.at[pl.ds(s * RPT2 + t * CK, CK)])
        plsc.subcore_barrier()
        for q in range(CH2 // SLAB2):
            base = s * CH2 + q * SLAB2
            pltpu.sync_copy(g2_h.at[pl.ds(base, SLAB2)], g2_v)
            pltpu.sync_copy(rows_h.at[pl.ds(base, SLAB2)], sidx_v)
            lax.fori_loop(0, SLAB2, initbody, None)
            lax.fori_loop(0, SLAB2 // 2, mainbody, None)
        plsc.subcore_barrier()
        for t in range(RPT2 // CK):
            sl = pl.ds(s * RPT2 + t * CK, CK)
            pltpu.sync_copy(acc_sh.at[sl], buf1)
            pltpu.sync_copy(buf1, out_h.at[strip, sl])


BM = 256


def _tca_body(x_b, cc_b, db_b, dc_b, orev, occ, a1_o, a2cc_o, a2rev_o):
    db = db_b[0] + db_b[1]
    dc = dc_b[0] + dc_b[1]
    cc = cc_b[...]
    x = x_b[...]
    a1_o[...] = lax.rsqrt(db + 1.0)
    occ[...] = x * lax.rsqrt(cc * db + 1.0)
    orev[...] = x * lax.rsqrt((1.0 - cc) * db + 1.0)
    a2cc_o[...] = lax.rsqrt(dc + 1.0)
    a2rev_o[...] = lax.rsqrt(db - dc + 1.0)


def _tcb_body(x_b, cc_b, a1_b, a2cc_b, a2rev_b, lamx_b, acc_b,
              w1l, w1h, w2l, w2h, wx, t2rev_o, t2cc_o, rest_o):
    x = x_b[...]
    cc = cc_b[...]
    a1 = a1_b[...]
    agg1 = acc_b[0] + acc_b[1]
    selden = a1 * agg1 + (a1 * a1) * x
    xlin = jnp.where(cc > 0.0, selden, x)
    xhin = jnp.where(cc > 0.0, x, selden)
    xl = jnp.maximum(jnp.dot(xlin, w1l[...], preferred_element_type=jnp.float32), 0.0)
    xh = jnp.maximum(jnp.dot(xhin, w1h[...], preferred_element_type=jnp.float32), 0.0)
    yl = jnp.dot(xl, w2l[...], preferred_element_type=jnp.float32)
    yh = jnp.dot(xh, w2h[...], preferred_element_type=jnp.float32)
    a2cc = a2cc_b[...]
    a2rev = a2rev_b[...]
    t2cc_o[...] = a2cc * yl
    t2rev_o[...] = a2rev * yh
    rest_o[...] = (lamx_b[...] * jnp.dot(x, wx[...], preferred_element_type=jnp.float32)
                   + (a2cc * a2cc) * yl + (a2rev * a2rev) * yh)


def _tcc_body(lo_b, hi_b, a2rev_b, a2cc_b, rest_b, lw_b, lb_b, out_o):
    lo = lo_b[...]
    hi = hi_b[...]
    pre = (a2rev_b[...] * jnp.concatenate([lo[0], lo[1], lo[2], lo[3]], axis=1)
           + a2cc_b[...] * jnp.concatenate([hi[0], hi[1], hi[2], hi[3]], axis=1)
           + rest_b[...])
    xf = jnp.maximum(pre, 0.0)
    lw = lw_b[...]
    out_o[...] = (jnp.dot(xf[:, :F], lw[:F, :], preferred_element_type=jnp.float32)
                  + jnp.dot(xf[:, F:], lw[F:, :], preferred_element_type=jnp.float32)
                  + lb_b[...])


def _col_spec(bm, width):
    return pl.BlockSpec((bm, width), lambda i: (i, 0))


def kernel(x, cc_mask, edge_index, W1L, W1H, W2L, W2H, WX, lam1, lam2, lin_w, lin_b):
    f32 = jnp.float32
    ei = edge_index.astype(jnp.int32)
    rows = jnp.pad(ei[1], (0, EP - E), constant_values=N).reshape(EG, CK)
    cols = jnp.pad(ei[0], (0, EP - E), constant_values=N).reshape(EG, CK)
    cc_p = jnp.pad(cc_mask.astype(f32), (0, NP - N))
    x_p = jnp.pad(x.astype(f32), ((0, NP - N), (0, 0)))

    g1, g2, degb_p, degc_p = _prep(rows, cols, cc_p)

    nblocks = NP // BM
    vec_spec = _col_spec(BM, 1)
    deg_spec = pl.BlockSpec((2, BM, 1), lambda i: (0, i, 0))
    orev, occ, a1, a2cc, a2rev = pl.pallas_call(
        _tca_body,
        grid=(nblocks,),
        in_specs=[_col_spec(BM, F), vec_spec, deg_spec, deg_spec],
        out_specs=[_col_spec(BM, F), _col_spec(BM, F), vec_spec, vec_spec, vec_spec],
        out_shape=[
            jax.ShapeDtypeStruct((NP, F), f32),
            jax.ShapeDtypeStruct((NP, F), f32),
            jax.ShapeDtypeStruct((NP, 1), f32),
            jax.ShapeDtypeStruct((NP, 1), f32),
            jax.ShapeDtypeStruct((NP, 1), f32),
        ],
    )(x_p, cc_p.reshape(NP, 1), degb_p.reshape(2, NP, 1), degc_p.reshape(2, NP, 1))

    t1 = jnp.concatenate([orev, occ, jnp.zeros((16, F), f32)], axis=0)
    acc1 = _spmm1(g1, rows, t1)

    l1 = jax.nn.softmax(lam1)
    l2 = jax.nn.softmax(lam2)
    lamx = (l1[0] * cc_p + l2[0] * (1.0 - cc_p)).reshape(NP, 1)
    w2l_s = W2L.astype(f32) * l1[1]
    w2h_s = W2H.astype(f32) * l2[1]

    full = lambda shape: pl.BlockSpec(shape, lambda i: tuple(0 for _ in shape))
    t2rev, t2cc, rest = pl.pallas_call(
        _tcb_body,
        grid=(nblocks,),
        in_specs=[
            _col_spec(BM, F), vec_spec, vec_spec, vec_spec, vec_spec, vec_spec,
            pl.BlockSpec((2, BM, F), lambda i: (0, i, 0)),
            full((F, H)), full((F, H)), full((H, H)), full((H, H)), full((F, H)),
        ],
        out_specs=[_col_spec(BM, H), _col_spec(BM, H), _col_spec(BM, H)],
        out_shape=[
            jax.ShapeDtypeStruct((NP, H), f32),
            jax.ShapeDtypeStruct((NP, H), f32),
            jax.ShapeDtypeStruct((NP, H), f32),
        ],
    )(x_p, cc_p.reshape(NP, 1), a1, a2cc, a2rev, lamx, acc1,
      W1L.astype(f32), W1H.astype(f32), w2l_s, w2h_s, WX.astype(f32))

    zs = jnp.zeros((16, FS), f32)
    parts = []
    for st in range(4):
        cs = slice(st * FS, (st + 1) * FS)
        parts += [t2rev[:, cs], t2cc[:, cs], zs]
    t2s = jnp.concatenate(parts, axis=0)

    acc2 = _spmm2(g2, rows, t2s)

    lw_pad = jnp.pad(lin_w.astype(f32), ((0, 0), (0, 128 - C)))
    lb_pad = jnp.pad(lin_b.astype(f32), (0, 128 - C)).reshape(1, 128)
    outp = pl.pallas_call(
        _tcc_body,
        grid=(nblocks,),
        in_specs=[pl.BlockSpec((4, BM, FS), lambda i: (0, i, 0)),
                  pl.BlockSpec((4, BM, FS), lambda i: (0, i + NP // BM, 0)),
                  vec_spec, vec_spec, _col_spec(BM, H),
                  full((H, 128)), full((1, 128))],
        out_specs=_col_spec(BM, 128),
        out_shape=jax.ShapeDtypeStruct((NP, 128), f32),
    )(acc2, acc2, a2rev, a2cc, rest, lw_pad, lb_pad)

    return outp[:N, :C]
